# baseline (speedup 1.0000x reference)
import dataclasses
import functools

import jax
import jax.numpy as jnp
from jax import lax
from jax.experimental import pallas as pl
from jax.experimental.pallas import tpu as pltpu
from jax.experimental.pallas import tpu_sc as plsc

N = 10000
E = 320000
D = 128
NC = 2
NS = 16
NW = NC * NS
EPW = E // NW

K = 80
C = EPW // K

DK = 2000
DC = EPW // DK

NP = 10240
SP = NP // NS
RZ = 128

_MESH = plsc.VectorSubcoreMesh(core_axis_name="c", subcore_axis_name="s")

_SC_PARAMS = pltpu.CompilerParams()
if "needs_layout_passes" in pltpu.CompilerParams.__dataclass_fields__:
    _SC_PARAMS = dataclasses.replace(_SC_PARAMS, needs_layout_passes=False)

_F32 = jnp.float32


@functools.partial(
    pl.kernel,
    out_type=jax.ShapeDtypeStruct((NW * N,), _F32),
    mesh=_MESH,
    compiler_params=_SC_PARAMS,
    scratch_types=[
        pltpu.VMEM((N,), _F32),
        pltpu.VMEM((DK,), jnp.int32),
        pltpu.VMEM((DK,), _F32),
    ],
)
def _sc_degree(col_hbm, ew_hbm, out_hbm, deg_v, col_v, ew_v):
    cid = lax.axis_index("c")
    sid = lax.axis_index("s")
    wid = sid * NC + cid
    base = wid * EPW

    @pl.loop(0, N, step=16)
    def _(i):
        deg_v[pl.ds(i, 16)] = jnp.zeros((16,), _F32)

    @pl.loop(0, DC)
    def _(t):
        pltpu.sync_copy(col_hbm.at[pl.ds(base + t * DK, DK)], col_v)
        pltpu.sync_copy(ew_hbm.at[pl.ds(base + t * DK, DK)], ew_v)

        @pl.loop(0, DK, step=16)
        def _(j):
            cv = col_v[pl.ds(j, 16)]
            wv = ew_v[pl.ds(j, 16)]
            plsc.addupdate_scatter(deg_v, [cv], wv)

    pltpu.sync_copy(deg_v, out_hbm.at[pl.ds(wid * N, N)])


@functools.partial(
    pl.kernel,
    out_type=jax.ShapeDtypeStruct((NC, NP, D), _F32),
    mesh=_MESH,
    compiler_params=_SC_PARAMS,
    scratch_types=[
        pltpu.VMEM_SHARED((NP, D), _F32),
        pltpu.VMEM((4, K, D), _F32),
        pltpu.VMEM((4, 2, K), jnp.int32),
        pltpu.VMEM((4, K), jnp.int32),
        pltpu.VMEM((4, K), jnp.int32),
        pltpu.SemaphoreType.DMA((4,)),
        pltpu.SemaphoreType.DMA((4,)),
        pltpu.SemaphoreType.DMA((4,)),
    ],
)
def _sc_msgpass(g_hbm, pkew_hbm, out_hbm,
                acc_sh, buf, idx, rowb, colb, sem_g, sem_i, sem_s):
    cid = lax.axis_index("c")
    sid = lax.axis_index("s")
    wid = sid * NC + cid
    gbase = wid * C

    def issue_idx(u, b):
        pltpu.async_copy(pkew_hbm.at[gbase + u], idx.at[b], sem_i.at[b])

    def wait_idx(b):
        pltpu.make_async_copy(pkew_hbm.at[gbase], idx.at[b], sem_i.at[b]).wait()

    def unpack(b):
        @pl.loop(0, K, step=16)
        def _(j):
            p = idx[b, 0, pl.ds(j, 16)]
            rowb[b, pl.ds(j, 16)] = lax.shift_right_logical(p, 14)
            colb[b, pl.ds(j, 16)] = lax.bitwise_and(p, 16383)

    def issue_gather(b):
        pltpu.async_copy(g_hbm.at[rowb.at[b]], buf.at[b], sem_g.at[b])

    def wait_gather(b):
        pltpu.make_async_copy(g_hbm.at[rowb.at[b]], buf.at[b],
                              sem_g.at[b]).wait()

    def scale(b):
        @pl.loop(0, K, step=16)
        def _(j):
            wv = plsc.bitcast(idx[b, 1, pl.ds(j, 16)], _F32)
            for jj in range(16):
                wgt = wv[jj]
                for t8 in range(D // 16):
                    sl = pl.ds(16 * t8, 16)
                    buf[b, j + jj, sl] = buf[b, j + jj, sl] * wgt

    def issue_scatter(b):
        pltpu.async_copy(buf.at[b], acc_sh.at[colb.at[b]], sem_s.at[b],
                         add=True)

    def wait_scatter(b):
        pltpu.make_async_copy(buf.at[b], acc_sh.at[colb.at[b]],
                              sem_s.at[b]).wait()

    @pl.loop(0, K)
    def _(j):
        for t8 in range(D // 16):
            buf[0, j, pl.ds(16 * t8, 16)] = jnp.zeros((16,), _F32)

    @pl.loop(0, SP, step=K)
    def _(r):
        pltpu.sync_copy(buf.at[0], acc_sh.at[pl.ds(sid * SP + r, K)])

    plsc.subcore_barrier()

    for b in range(4):
        issue_idx(b, b)
    for b in range(3):
        wait_idx(b)
        unpack(b)
        issue_gather(b)

    @pl.loop(0, C - 5, step=4)
    def _(t):
        for off in range(4):
            b = off
            u = t + off
            wait_gather(b)
            scale(b)
            issue_scatter(b)
            b3 = (b + 3) % 4
            if off == 0:
                @pl.when(t > 0)
                def _():
                    wait_scatter(b3)
            else:
                wait_scatter(b3)
            wait_idx(b3)
            unpack(b3)
            issue_gather(b3)
            issue_idx(u + 4, b)

    wait_gather(0)
    scale(0)
    issue_scatter(0)
    issue_idx(C - 1, 0)
    wait_scatter(3)
    wait_idx(3)
    unpack(3)
    issue_gather(3)
    wait_gather(1)
    scale(1)
    issue_scatter(1)
    wait_scatter(0)
    wait_idx(0)
    unpack(0)
    issue_gather(0)
    wait_gather(2)
    scale(2)
    issue_scatter(2)
    wait_scatter(1)
    wait_gather(3)
    scale(3)
    issue_scatter(3)
    wait_scatter(2)
    wait_gather(0)
    scale(0)
    issue_scatter(0)
    wait_scatter(3)
    wait_scatter(0)

    plsc.subcore_barrier()

    @pl.loop(0, SP, step=RZ)
    def _(r):
        pltpu.sync_copy(acc_sh.at[pl.ds(sid * SP + r, RZ)],
                        out_hbm.at[cid, pl.ds(sid * SP + r, RZ)])


def _mm_body(x_ref, w_ref, o_ref):
    o_ref[...] = jnp.dot(x_ref[...], w_ref[...], preferred_element_type=_F32)


def _tc_matmul(x, w):
    return pl.pallas_call(
        _mm_body,
        out_shape=jax.ShapeDtypeStruct((x.shape[0], w.shape[1]), _F32),
    )(x, w)


def _prep_body(parts_ref, x_ref, w1_ref, dis_ref, h1_ref, g1_ref):
    deg = jnp.sum(parts_ref[...].T, axis=1, keepdims=True) + 1.0
    dis = lax.rsqrt(deg)
    dis_ref[...] = dis
    h1 = jnp.dot(x_ref[...], w1_ref[...], preferred_element_type=_F32)
    h1_ref[...] = h1
    g1_ref[...] = h1 * dis


def _tc_prep(parts, x, w1):
    return pl.pallas_call(
        _prep_body,
        out_shape=(
            jax.ShapeDtypeStruct((N, 1), _F32),
            jax.ShapeDtypeStruct((N, D), _F32),
            jax.ShapeDtypeStruct((N, D), _F32),
        ),
    )(parts, x, w1)


def _mid_body(p_ref, dis_ref, h1_ref, b1_ref, w2_ref, h2_ref, g2_ref):
    dis = dis_ref[...]
    acc = p_ref[0, :N, :] + p_ref[1, :N, :]
    z = dis * acc + (dis * dis) * h1_ref[...] + b1_ref[...][None, :]
    z = jnp.maximum(z, 0.0)
    h2 = jnp.dot(z, w2_ref[...], preferred_element_type=_F32)
    h2_ref[...] = h2
    g2_ref[...] = h2 * dis


def _tc_mid(p1, dis, h1, b1, w2):
    return pl.pallas_call(
        _mid_body,
        out_shape=(
            jax.ShapeDtypeStruct((N, D), _F32),
            jax.ShapeDtypeStruct((N, D), _F32),
        ),
    )(p1, dis, h1, b1, w2)


def _final_body(p_ref, dis_ref, h2_ref, b2_ref, o_ref):
    dis = dis_ref[...]
    acc = p_ref[0, :N, :] + p_ref[1, :N, :]
    o_ref[...] = dis * acc + (dis * dis) * h2_ref[...] + b2_ref[...][None, :]


def _tc_final(p2, dis, h2, b2):
    return pl.pallas_call(
        _final_body,
        out_shape=jax.ShapeDtypeStruct((N, D), _F32),
    )(p2, dis, h2, b2)


def kernel(x, edge_index, edge_weight, W1, b1, W2, b2):
    row = edge_index[0]
    col = edge_index[1]

    packed = jnp.bitwise_or(jnp.left_shift(row, 14), col)
    ew_bits = lax.bitcast_convert_type(edge_weight, jnp.int32)
    pkew = jnp.stack(
        [packed.reshape(E // K, K), ew_bits.reshape(E // K, K)], axis=1)

    deg_flat = _sc_degree(col, edge_weight)
    dis, h1, g1 = _tc_prep(deg_flat.reshape(NW, N), x, W1)
    p1 = _sc_msgpass(g1, pkew)
    h2, g2 = _tc_mid(p1, dis, h1, b1, W2)
    p2 = _sc_msgpass(g2, pkew)
    return _tc_final(p2, dis, h2, b2)


# speedup vs baseline: 1.0027x; 1.0027x over previous
import dataclasses
import functools

import jax
import jax.numpy as jnp
from jax import lax
from jax.experimental import pallas as pl
from jax.experimental.pallas import tpu as pltpu
from jax.experimental.pallas import tpu_sc as plsc

N = 10000
E = 320000
D = 128
NC = 2
NS = 16
NW = NC * NS
EPW = E // NW

K = 80
C = EPW // K

DK = 2000
DC = EPW // DK

NP = 10240
SP = NP // NS
RZ = 128

_MESH = plsc.VectorSubcoreMesh(core_axis_name="c", subcore_axis_name="s")

_SC_PARAMS = pltpu.CompilerParams()
if "needs_layout_passes" in pltpu.CompilerParams.__dataclass_fields__:
    _SC_PARAMS = dataclasses.replace(_SC_PARAMS, needs_layout_passes=False)

_F32 = jnp.float32


@functools.partial(
    pl.kernel,
    out_type=jax.ShapeDtypeStruct((NW * N,), _F32),
    mesh=_MESH,
    compiler_params=_SC_PARAMS,
    scratch_types=[
        pltpu.VMEM((N,), _F32),
        pltpu.VMEM((DK,), jnp.int32),
        pltpu.VMEM((DK,), _F32),
    ],
)
def _sc_degree(col_hbm, ew_hbm, out_hbm, deg_v, col_v, ew_v):
    cid = lax.axis_index("c")
    sid = lax.axis_index("s")
    wid = sid * NC + cid
    base = wid * EPW

    @pl.loop(0, N, step=16)
    def _(i):
        deg_v[pl.ds(i, 16)] = jnp.zeros((16,), _F32)

    @pl.loop(0, DC)
    def _(t):
        pltpu.sync_copy(col_hbm.at[pl.ds(base + t * DK, DK)], col_v)
        pltpu.sync_copy(ew_hbm.at[pl.ds(base + t * DK, DK)], ew_v)

        @pl.loop(0, DK, step=16)
        def _(j):
            cv = col_v[pl.ds(j, 16)]
            wv = ew_v[pl.ds(j, 16)]
            plsc.addupdate_scatter(deg_v, [cv], wv)

    pltpu.sync_copy(deg_v, out_hbm.at[pl.ds(wid * N, N)])


@functools.partial(
    pl.kernel,
    out_type=jax.ShapeDtypeStruct((NC, NP, D), _F32),
    mesh=_MESH,
    compiler_params=_SC_PARAMS,
    scratch_types=[
        pltpu.VMEM_SHARED((NP, D), _F32),
        pltpu.VMEM((4, K, D), _F32),
        pltpu.VMEM((4, 2, K), jnp.int32),
        pltpu.VMEM((4, K), jnp.int32),
        pltpu.VMEM((4, K), jnp.int32),
        pltpu.SemaphoreType.DMA((4,)),
        pltpu.SemaphoreType.DMA((4,)),
        pltpu.SemaphoreType.DMA((4,)),
    ],
)
def _sc_msgpass(g_hbm, pkew_hbm, out_hbm,
                acc_sh, buf, idx, rowb, colb, sem_g, sem_i, sem_s):
    cid = lax.axis_index("c")
    sid = lax.axis_index("s")
    wid = sid * NC + cid
    gbase = wid * C

    def issue_idx(u, b):
        pltpu.async_copy(pkew_hbm.at[gbase + u], idx.at[b], sem_i.at[b])

    def wait_idx(b):
        pltpu.make_async_copy(pkew_hbm.at[gbase], idx.at[b], sem_i.at[b]).wait()

    def unpack(b):
        @pl.loop(0, K, step=16)
        def _(j):
            p = idx[b, 0, pl.ds(j, 16)]
            rowb[b, pl.ds(j, 16)] = lax.shift_right_logical(p, 14)
            colb[b, pl.ds(j, 16)] = lax.bitwise_and(p, 16383)

    def issue_gather(b):
        pltpu.async_copy(g_hbm.at[rowb.at[b]], buf.at[b], sem_g.at[b])

    def wait_gather(b):
        pltpu.make_async_copy(g_hbm.at[rowb.at[b]], buf.at[b],
                              sem_g.at[b]).wait()

    def scale(b):
        @pl.loop(0, K, step=16)
        def _(j):
            wv = plsc.bitcast(idx[b, 1, pl.ds(j, 16)], _F32)
            for jj in range(16):
                wgt = wv[jj]
                for t8 in range(D // 16):
                    sl = pl.ds(16 * t8, 16)
                    buf[b, j + jj, sl] = buf[b, j + jj, sl] * wgt

    def issue_scatter(b):
        pltpu.async_copy(buf.at[b], acc_sh.at[colb.at[b]], sem_s.at[b],
                         add=True)

    def wait_scatter(b):
        pltpu.make_async_copy(buf.at[b], acc_sh.at[colb.at[b]],
                              sem_s.at[b]).wait()

    @pl.loop(0, K)
    def _(j):
        for t8 in range(D // 16):
            buf[0, j, pl.ds(16 * t8, 16)] = jnp.zeros((16,), _F32)

    @pl.loop(0, SP, step=K)
    def _(r):
        pltpu.sync_copy(buf.at[0], acc_sh.at[pl.ds(sid * SP + r, K)])

    plsc.subcore_barrier()

    for b in range(4):
        issue_idx(b, b)
    for b in range(3):
        wait_idx(b)
        unpack(b)
        issue_gather(b)

    @pl.loop(0, C - 5, step=4)
    def _(t):
        for off in range(4):
            b = off
            u = t + off
            wait_gather(b)
            scale(b)
            issue_scatter(b)
            b3 = (b + 3) % 4
            if off == 0:
                @pl.when(t > 0)
                def _():
                    wait_scatter(b3)
            else:
                wait_scatter(b3)
            wait_idx(b3)
            unpack(b3)
            issue_gather(b3)
            issue_idx(u + 4, b)

    wait_gather(0)
    scale(0)
    issue_scatter(0)
    issue_idx(C - 1, 0)
    wait_scatter(3)
    wait_idx(3)
    unpack(3)
    issue_gather(3)
    wait_gather(1)
    scale(1)
    issue_scatter(1)
    wait_scatter(0)
    wait_idx(0)
    unpack(0)
    issue_gather(0)
    wait_gather(2)
    scale(2)
    issue_scatter(2)
    wait_scatter(1)
    wait_gather(3)
    scale(3)
    issue_scatter(3)
    wait_scatter(2)
    wait_gather(0)
    scale(0)
    issue_scatter(0)
    wait_scatter(3)
    wait_scatter(0)

    plsc.subcore_barrier()

    @pl.loop(0, SP, step=RZ)
    def _(r):
        pltpu.sync_copy(acc_sh.at[pl.ds(sid * SP + r, RZ)],
                        out_hbm.at[cid, pl.ds(sid * SP + r, RZ)])


def _mm_body(x_ref, w_ref, o_ref):
    o_ref[...] = jnp.dot(x_ref[...], w_ref[...], preferred_element_type=_F32)


def _tc_matmul(x, w):
    return pl.pallas_call(
        _mm_body,
        out_shape=jax.ShapeDtypeStruct((x.shape[0], w.shape[1]), _F32),
    )(x, w)


def _prep_body(parts_ref, h1_ref, dis_ref, g1_ref):
    deg = jnp.sum(parts_ref[...].T, axis=1, keepdims=True) + 1.0
    dis = lax.rsqrt(deg)
    dis_ref[...] = dis
    g1_ref[...] = h1_ref[...] * dis


def _tc_prep(parts, h1):
    return pl.pallas_call(
        _prep_body,
        out_shape=(
            jax.ShapeDtypeStruct((N, 1), _F32),
            jax.ShapeDtypeStruct((N, D), _F32),
        ),
    )(parts, h1)


def _mid_body(p_ref, dis_ref, h1_ref, b1_ref, w2_ref, h2_ref, g2_ref):
    dis = dis_ref[...]
    acc = p_ref[0, :N, :] + p_ref[1, :N, :]
    z = dis * acc + (dis * dis) * h1_ref[...] + b1_ref[...][None, :]
    z = jnp.maximum(z, 0.0)
    h2 = jnp.dot(z, w2_ref[...], preferred_element_type=_F32)
    h2_ref[...] = h2
    g2_ref[...] = h2 * dis


def _tc_mid(p1, dis, h1, b1, w2):
    return pl.pallas_call(
        _mid_body,
        out_shape=(
            jax.ShapeDtypeStruct((N, D), _F32),
            jax.ShapeDtypeStruct((N, D), _F32),
        ),
    )(p1, dis, h1, b1, w2)


def _final_body(p_ref, dis_ref, h2_ref, b2_ref, o_ref):
    dis = dis_ref[...]
    acc = p_ref[0, :N, :] + p_ref[1, :N, :]
    o_ref[...] = dis * acc + (dis * dis) * h2_ref[...] + b2_ref[...][None, :]


def _tc_final(p2, dis, h2, b2):
    return pl.pallas_call(
        _final_body,
        out_shape=jax.ShapeDtypeStruct((N, D), _F32),
    )(p2, dis, h2, b2)


def kernel(x, edge_index, edge_weight, W1, b1, W2, b2):
    row = edge_index[0]
    col = edge_index[1]

    packed = jnp.bitwise_or(jnp.left_shift(row, 14), col)
    ew_bits = lax.bitcast_convert_type(edge_weight, jnp.int32)
    pkew = jnp.stack(
        [packed.reshape(E // K, K), ew_bits.reshape(E // K, K)], axis=1)

    deg_flat = _sc_degree(col, edge_weight)
    h1 = _tc_matmul(x, W1)
    dis, g1 = _tc_prep(deg_flat.reshape(NW, N), h1)
    p1 = _sc_msgpass(g1, pkew)
    h2, g2 = _tc_mid(p1, dis, h1, b1, W2)
    p2 = _sc_msgpass(g2, pkew)
    return _tc_final(p2, dis, h2, b2)
